# Initial kernel scaffold; baseline (speedup 1.0000x reference)
import jax
import jax.numpy as jnp
from jax.experimental import pallas as pl


def kernel(x, edge_index, batch, W1, b1, W2, b2, W3, b3, Wl, bl):
    raise NotImplementedError("write your pallas kernel here")


# speedup vs baseline: 8.4148x; 8.4148x over previous
import functools

import jax
import jax.numpy as jnp
from jax import lax
from jax.experimental import pallas as pl
from jax.experimental.pallas import tpu as pltpu
from jax.experimental.pallas import tpu_sc as plsc

NN = 10000
NP = 10240
EE = 160000
GG = 64
INCH = 3000
HID = 256
OUT = 128
NT = 16
CH = 80
NCH = 125
BN = 1024

_mesh = plsc.VectorSubcoreMesh(core_axis_name="c", subcore_axis_name="s")


def _deg_body(dst_hbm, deg_hbm, dst_v, degv):
    c = lax.axis_index("c")
    s = lax.axis_index("s")

    @pl.when(c == 0)
    def _():
        def zz(i, carry):
            degv[pl.ds(i * 16, 16)] = jnp.zeros((16,), jnp.float32)
            return carry

        lax.fori_loop(0, NP // 16, zz, 0)
        pltpu.sync_copy(dst_hbm.at[s], dst_v)
        ones = jnp.ones((16,), jnp.float32)

        def row(j, carry):
            drow = dst_v.at[j]
            for k in range(CH // 16):
                idx = drow[pl.ds(k * 16, 16)]
                plsc.addupdate_scatter(degv, [idx], ones)
            return carry

        lax.fori_loop(0, NCH, row, 0)
        pltpu.sync_copy(degv, deg_hbm.at[s])


_deg_call = pl.kernel(
    _deg_body,
    out_type=jax.ShapeDtypeStruct((NT, NP), jnp.float32),
    mesh=_mesh,
    scratch_types=[
        pltpu.VMEM((NCH, CH), jnp.int32),
        pltpu.VMEM((NP,), jnp.float32),
    ],
    compiler_params=pltpu.CompilerParams(needs_layout_passes=False),
)


def _agg_body(split, g_hbm, src_hbm, dst_hbm, out_hbm,
              src_v, dst_v, dst2_v, buf0, buf1, acc, sem0, sem1):
    c = lax.axis_index("c")
    s = lax.axis_index("s")
    hr = 5120 if split else 2560
    zr = hr // NT
    ghalf = g_hbm.at[c] if split else g_hbm

    pltpu.sync_copy(src_hbm.at[s], src_v)
    pltpu.sync_copy(dst_hbm.at[s], dst_v)

    for p in (0, 1):
        lo = hr * p if split else hr * (2 * c + p)

        def zrow(i, carry):
            brow = buf0.at[i]
            for j in range(128 // 16):
                brow[pl.ds(j * 16, 16)] = jnp.zeros((16,), jnp.float32)
            return carry

        lax.fori_loop(0, CH, zrow, 0)
        for z in range(zr // CH):
            pltpu.sync_copy(buf0, acc.at[pl.ds(s * zr + z * CH, CH)])

        @pl.when(s < 8)
        def _():
            pltpu.sync_copy(buf0.at[pl.ds(0, 1)], acc.at[pl.ds(hr + s, 1)])

        def rrow(j, carry):
            drow = dst_v.at[j]
            d2row = dst2_v.at[j]
            for k in range(CH // 16):
                d = drow[pl.ds(k * 16, 16)]
                inr = (d >= lo) & (d < lo + hr)
                dummy = hr + ((s + k) & 7)
                d2row[pl.ds(k * 16, 16)] = jnp.where(
                    inr, d - lo, jnp.zeros((16,), jnp.int32) + dummy)
            return carry

        lax.fori_loop(0, NCH, rrow, 0)
        plsc.subcore_barrier()

        def fire(j, buf, sem):
            pltpu.async_copy(ghalf.at[src_v.at[j]], buf, sem)

        def wait(buf, sem):
            pltpu.make_async_copy(ghalf.at[src_v.at[0]], buf, sem).wait()

        def scat(j, buf):
            pltpu.sync_copy(buf, acc.at[dst2_v.at[j]], add=True)

        fire(0, buf0, sem0)
        fire(1, buf1, sem1)

        def pair(i, carry):
            wait(buf0, sem0)
            scat(2 * i, buf0)
            fire(2 * i + 2, buf0, sem0)
            wait(buf1, sem1)
            scat(2 * i + 1, buf1)
            fire(2 * i + 3, buf1, sem1)
            return carry

        lax.fori_loop(0, (NCH - 3) // 2, pair, 0)
        wait(buf0, sem0)
        scat(NCH - 3, buf0)
        fire(NCH - 1, buf0, sem0)
        wait(buf1, sem1)
        scat(NCH - 2, buf1)
        wait(buf0, sem0)
        scat(NCH - 1, buf0)

        plsc.subcore_barrier()
        if split:
            odst = out_hbm.at[c, pl.ds(lo + s * zr, zr)]
        else:
            odst = out_hbm.at[pl.ds(lo + s * zr, zr)]
        pltpu.sync_copy(acc.at[pl.ds(s * zr, zr)], odst)
        plsc.subcore_barrier()


def _mk_agg(split):
    hr = 5120 if split else 2560
    oshape = (2, NP, 128) if split else (NP, 128)
    return pl.kernel(
        functools.partial(_agg_body, split),
        out_type=jax.ShapeDtypeStruct(oshape, jnp.float32),
        mesh=_mesh,
        scratch_types=[
            pltpu.VMEM((NCH, CH), jnp.int32),
            pltpu.VMEM((NCH, CH), jnp.int32),
            pltpu.VMEM((NCH, CH), jnp.int32),
            pltpu.VMEM((CH, 128), jnp.float32),
            pltpu.VMEM((CH, 128), jnp.float32),
            pltpu.VMEM_SHARED((hr + 8, 128), jnp.float32),
            pltpu.SemaphoreType.DMA,
            pltpu.SemaphoreType.DMA,
        ],
    )


_agg_half = _mk_agg(True)
_agg_full = _mk_agg(False)


def _k1_body(deg16_ref, x_ref, w_ref, g_ref, deg_ref):
    i = pl.program_id(0)
    deg = jnp.sum(deg16_ref[...], axis=0)
    deg_ref[...] = deg
    dinv = lax.rsqrt(deg + 1.0)[:, None]
    t = jnp.dot(x_ref[...], w_ref[...], preferred_element_type=jnp.float32)
    row = i * BN + lax.broadcasted_iota(jnp.int32, (BN, 1), 0)
    g = jnp.where(row < NN, t * dinv, 0.0)
    g_ref[0] = g[:, :128]
    g_ref[1] = g[:, 128:]


_k1 = pl.pallas_call(
    _k1_body,
    grid=(NP // BN,),
    in_specs=[
        pl.BlockSpec((NT, BN), lambda i: (0, i)),
        pl.BlockSpec((BN, INCH), lambda i: (i, 0)),
        pl.BlockSpec((INCH, HID), lambda i: (0, 0)),
    ],
    out_specs=[
        pl.BlockSpec((2, BN, 128), lambda i: (0, i, 0)),
        pl.BlockSpec((BN,), lambda i: (i,)),
    ],
    out_shape=[
        jax.ShapeDtypeStruct((2, NP, 128), jnp.float32),
        jax.ShapeDtypeStruct((NP,), jnp.float32),
    ],
)


def _k2_body(deg_ref, s_ref, g_ref, b_ref, w_ref, out_ref):
    dinv = lax.rsqrt(deg_ref[...] + 1.0)[:, None]
    h = jnp.concatenate([s_ref[0] + g_ref[0], s_ref[1] + g_ref[1]], axis=1)
    h = jnp.maximum(h * dinv + b_ref[...][None, :], 0.0)
    t = jnp.dot(h, w_ref[...], preferred_element_type=jnp.float32)
    g = t * dinv
    out_ref[0] = g[:, :128]
    out_ref[1] = g[:, 128:]


_k2 = pl.pallas_call(
    _k2_body,
    grid=(NP // BN,),
    in_specs=[
        pl.BlockSpec((BN,), lambda i: (i,)),
        pl.BlockSpec((2, BN, 128), lambda i: (0, i, 0)),
        pl.BlockSpec((2, BN, 128), lambda i: (0, i, 0)),
        pl.BlockSpec((HID,), lambda i: (0,)),
        pl.BlockSpec((HID, HID), lambda i: (0, 0)),
    ],
    out_specs=pl.BlockSpec((2, BN, 128), lambda i: (0, i, 0)),
    out_shape=jax.ShapeDtypeStruct((2, NP, 128), jnp.float32),
)


def _k3_body(deg_ref, s_ref, g_ref, b_ref, w_ref, out_ref):
    dinv = lax.rsqrt(deg_ref[...] + 1.0)[:, None]
    h = jnp.concatenate([s_ref[0] + g_ref[0], s_ref[1] + g_ref[1]], axis=1)
    h = jnp.maximum(h * dinv + b_ref[...][None, :], 0.0)
    t = jnp.dot(h, w_ref[...], preferred_element_type=jnp.float32)
    out_ref[...] = t * dinv


_k3 = pl.pallas_call(
    _k3_body,
    grid=(NP // BN,),
    in_specs=[
        pl.BlockSpec((BN,), lambda i: (i,)),
        pl.BlockSpec((2, BN, 128), lambda i: (0, i, 0)),
        pl.BlockSpec((2, BN, 128), lambda i: (0, i, 0)),
        pl.BlockSpec((HID,), lambda i: (0,)),
        pl.BlockSpec((HID, OUT), lambda i: (0, 0)),
    ],
    out_specs=pl.BlockSpec((BN, OUT), lambda i: (i, 0)),
    out_shape=jax.ShapeDtypeStruct((NP, OUT), jnp.float32),
)


def _k4_body(deg_ref, s_ref, g_ref, b_ref, batch_ref, wl_ref, bl_ref, out_ref):
    dinv = lax.rsqrt(deg_ref[...] + 1.0)[:, None]
    h = jnp.maximum((s_ref[...] + g_ref[...]) * dinv + b_ref[...][None, :],
                    0.0)
    seg = lax.broadcasted_iota(jnp.int32, (GG, NP), 0)
    P = (batch_ref[...][None, :] == seg).astype(jnp.float32)
    sums = jnp.dot(P, h, preferred_element_type=jnp.float32)
    counts = jnp.sum(P, axis=1, keepdims=True)
    pooled = sums / jnp.maximum(counts, 1.0)
    out_ref[...] = (jnp.dot(pooled, wl_ref[...],
                            preferred_element_type=jnp.float32)
                    + bl_ref[...][None, :])


_k4 = pl.pallas_call(
    _k4_body,
    out_shape=jax.ShapeDtypeStruct((GG, 1), jnp.float32),
)


def kernel(x, edge_index, batch, W1, b1, W2, b2, W3, b3, Wl, bl):
    src = edge_index[0].reshape(NT, NCH, CH)
    dst = edge_index[1].reshape(NT, NCH, CH)
    batch_ext = jnp.concatenate(
        [batch, jnp.full((NP - NN,), GG, dtype=batch.dtype)])
    deg16 = _deg_call(dst)
    g1, deg = _k1(deg16, x, W1)
    s1 = _agg_half(g1, src, dst)
    g2 = _k2(deg, s1, g1, b1, W2)
    s2 = _agg_half(g2, src, dst)
    g3 = _k3(deg, s2, g2, b2, W3)
    s3 = _agg_full(g3, src, dst)
    return _k4(deg, s3, g3, b3, batch_ext, Wl, bl)


# speedup vs baseline: 8.4267x; 1.0014x over previous
import functools

import jax
import jax.numpy as jnp
from jax import lax
from jax.experimental import pallas as pl
from jax.experimental.pallas import tpu as pltpu
from jax.experimental.pallas import tpu_sc as plsc

NN = 10000
NP = 10240
EE = 160000
GG = 64
INCH = 3000
HID = 256
OUT = 128
NT = 16
CH = 80
NCH = 125
BN = 1024

_mesh = plsc.VectorSubcoreMesh(core_axis_name="c", subcore_axis_name="s")


def _deg_body(dst_hbm, deg_hbm, dst_v, degv):
    c = lax.axis_index("c")
    s = lax.axis_index("s")

    @pl.when(c == 0)
    def _():
        def zz(i, carry):
            degv[pl.ds(i * 16, 16)] = jnp.zeros((16,), jnp.float32)
            return carry

        lax.fori_loop(0, NP // 16, zz, 0)
        pltpu.sync_copy(dst_hbm.at[s], dst_v)
        ones = jnp.ones((16,), jnp.float32)

        def row(j, carry):
            drow = dst_v.at[j]
            for k in range(CH // 16):
                idx = drow[pl.ds(k * 16, 16)]
                plsc.addupdate_scatter(degv, [idx], ones)
            return carry

        lax.fori_loop(0, NCH, row, 0)
        pltpu.sync_copy(degv, deg_hbm.at[s])


_deg_call = pl.kernel(
    _deg_body,
    out_type=jax.ShapeDtypeStruct((NT, NP), jnp.float32),
    mesh=_mesh,
    scratch_types=[
        pltpu.VMEM((NCH, CH), jnp.int32),
        pltpu.VMEM((NP,), jnp.float32),
    ],
    compiler_params=pltpu.CompilerParams(needs_layout_passes=False),
)


def _agg_body(split, g_hbm, src_hbm, dst_hbm, out_hbm,
              src_v, dst_v, dst2_v, buf0, buf1, acc, sem0, sem1):
    c = lax.axis_index("c")
    s = lax.axis_index("s")
    hr = 5120 if split else 2560
    zr = hr // NT
    ghalf = g_hbm.at[c] if split else g_hbm

    pltpu.sync_copy(src_hbm.at[s], src_v)
    pltpu.sync_copy(dst_hbm.at[s], dst_v)

    for p in (0, 1):
        lo = hr * p if split else hr * (2 * c + p)

        def zrow(i, carry):
            brow = buf0.at[i]
            for j in range(128 // 16):
                brow[pl.ds(j * 16, 16)] = jnp.zeros((16,), jnp.float32)
            return carry

        lax.fori_loop(0, CH, zrow, 0)
        for z in range(zr // CH):
            pltpu.sync_copy(buf0, acc.at[pl.ds(s * zr + z * CH, CH)])

        @pl.when(s < 8)
        def _():
            pltpu.sync_copy(buf0.at[pl.ds(0, 1)], acc.at[pl.ds(hr + s, 1)])

        def rrow(j, carry):
            drow = dst_v.at[j]
            d2row = dst2_v.at[j]
            for k in range(CH // 16):
                d = drow[pl.ds(k * 16, 16)]
                inr = (d >= lo) & (d < lo + hr)
                dummy = hr + ((s + k) & 7)
                d2row[pl.ds(k * 16, 16)] = jnp.where(
                    inr, d - lo, jnp.zeros((16,), jnp.int32) + dummy)
            return carry

        lax.fori_loop(0, NCH, rrow, 0)
        plsc.subcore_barrier()

        def fire(j, buf, sem):
            pltpu.async_copy(ghalf.at[src_v.at[j]], buf, sem)

        def wait(buf, sem):
            pltpu.make_async_copy(ghalf.at[src_v.at[0]], buf, sem).wait()

        def scat(j, buf):
            pltpu.sync_copy(buf, acc.at[dst2_v.at[j]], add=True)

        fire(0, buf0, sem0)
        fire(1, buf1, sem1)

        def pair(i, carry):
            wait(buf0, sem0)
            scat(2 * i, buf0)
            fire(2 * i + 2, buf0, sem0)
            wait(buf1, sem1)
            scat(2 * i + 1, buf1)
            fire(2 * i + 3, buf1, sem1)
            return carry

        lax.fori_loop(0, (NCH - 3) // 2, pair, 0)
        wait(buf0, sem0)
        scat(NCH - 3, buf0)
        fire(NCH - 1, buf0, sem0)
        wait(buf1, sem1)
        scat(NCH - 2, buf1)
        wait(buf0, sem0)
        scat(NCH - 1, buf0)

        plsc.subcore_barrier()
        if split:
            odst = out_hbm.at[c, pl.ds(lo + s * zr, zr)]
        else:
            odst = out_hbm.at[pl.ds(lo + s * zr, zr)]
        pltpu.sync_copy(acc.at[pl.ds(s * zr, zr)], odst)
        plsc.subcore_barrier()


def _mk_agg(split):
    hr = 5120 if split else 2560
    oshape = (2, NP, 128) if split else (NP, 128)
    return pl.kernel(
        functools.partial(_agg_body, split),
        out_type=jax.ShapeDtypeStruct(oshape, jnp.float32),
        mesh=_mesh,
        scratch_types=[
            pltpu.VMEM((NCH, CH), jnp.int32),
            pltpu.VMEM((NCH, CH), jnp.int32),
            pltpu.VMEM((NCH, CH), jnp.int32),
            pltpu.VMEM((CH, 128), jnp.float32),
            pltpu.VMEM((CH, 128), jnp.float32),
            pltpu.VMEM_SHARED((hr + 8, 128), jnp.float32),
            pltpu.SemaphoreType.DMA,
            pltpu.SemaphoreType.DMA,
        ],
    )


_agg_half = _mk_agg(True)
_agg_full = _mk_agg(False)


def _k1_body(deg16_ref, x_ref, w_ref, g_ref, deg_ref):
    i = pl.program_id(0)
    deg = jnp.sum(deg16_ref[...], axis=0)
    deg_ref[...] = deg
    dinv = lax.rsqrt(deg + 1.0)[:, None]
    t = jnp.dot(x_ref[...].astype(jnp.bfloat16),
                w_ref[...].astype(jnp.bfloat16),
                preferred_element_type=jnp.float32)
    row = i * BN + lax.broadcasted_iota(jnp.int32, (BN, 1), 0)
    g = jnp.where(row < NN, t * dinv, 0.0)
    g_ref[0] = g[:, :128]
    g_ref[1] = g[:, 128:]


_k1 = pl.pallas_call(
    _k1_body,
    grid=(NP // BN,),
    in_specs=[
        pl.BlockSpec((NT, BN), lambda i: (0, i)),
        pl.BlockSpec((BN, INCH), lambda i: (i, 0)),
        pl.BlockSpec((INCH, HID), lambda i: (0, 0)),
    ],
    out_specs=[
        pl.BlockSpec((2, BN, 128), lambda i: (0, i, 0)),
        pl.BlockSpec((BN,), lambda i: (i,)),
    ],
    out_shape=[
        jax.ShapeDtypeStruct((2, NP, 128), jnp.float32),
        jax.ShapeDtypeStruct((NP,), jnp.float32),
    ],
)


def _k2_body(deg_ref, s_ref, g_ref, b_ref, w_ref, out_ref):
    dinv = lax.rsqrt(deg_ref[...] + 1.0)[:, None]
    h = jnp.concatenate([s_ref[0] + g_ref[0], s_ref[1] + g_ref[1]], axis=1)
    h = jnp.maximum(h * dinv + b_ref[...][None, :], 0.0)
    t = jnp.dot(h, w_ref[...], preferred_element_type=jnp.float32)
    g = t * dinv
    out_ref[0] = g[:, :128]
    out_ref[1] = g[:, 128:]


_k2 = pl.pallas_call(
    _k2_body,
    grid=(NP // BN,),
    in_specs=[
        pl.BlockSpec((BN,), lambda i: (i,)),
        pl.BlockSpec((2, BN, 128), lambda i: (0, i, 0)),
        pl.BlockSpec((2, BN, 128), lambda i: (0, i, 0)),
        pl.BlockSpec((HID,), lambda i: (0,)),
        pl.BlockSpec((HID, HID), lambda i: (0, 0)),
    ],
    out_specs=pl.BlockSpec((2, BN, 128), lambda i: (0, i, 0)),
    out_shape=jax.ShapeDtypeStruct((2, NP, 128), jnp.float32),
)


def _k3_body(deg_ref, s_ref, g_ref, b_ref, w_ref, out_ref):
    dinv = lax.rsqrt(deg_ref[...] + 1.0)[:, None]
    h = jnp.concatenate([s_ref[0] + g_ref[0], s_ref[1] + g_ref[1]], axis=1)
    h = jnp.maximum(h * dinv + b_ref[...][None, :], 0.0)
    t = jnp.dot(h, w_ref[...], preferred_element_type=jnp.float32)
    out_ref[...] = t * dinv


_k3 = pl.pallas_call(
    _k3_body,
    grid=(NP // BN,),
    in_specs=[
        pl.BlockSpec((BN,), lambda i: (i,)),
        pl.BlockSpec((2, BN, 128), lambda i: (0, i, 0)),
        pl.BlockSpec((2, BN, 128), lambda i: (0, i, 0)),
        pl.BlockSpec((HID,), lambda i: (0,)),
        pl.BlockSpec((HID, OUT), lambda i: (0, 0)),
    ],
    out_specs=pl.BlockSpec((BN, OUT), lambda i: (i, 0)),
    out_shape=jax.ShapeDtypeStruct((NP, OUT), jnp.float32),
)


def _k4_body(deg_ref, s_ref, g_ref, b_ref, batch_ref, wl_ref, bl_ref, out_ref):
    dinv = lax.rsqrt(deg_ref[...] + 1.0)[:, None]
    h = jnp.maximum((s_ref[...] + g_ref[...]) * dinv + b_ref[...][None, :],
                    0.0)
    seg = lax.broadcasted_iota(jnp.int32, (GG, NP), 0)
    P = (batch_ref[...][None, :] == seg).astype(jnp.float32)
    sums = jnp.dot(P, h, preferred_element_type=jnp.float32)
    counts = jnp.sum(P, axis=1, keepdims=True)
    pooled = sums / jnp.maximum(counts, 1.0)
    out_ref[...] = (jnp.dot(pooled, wl_ref[...],
                            preferred_element_type=jnp.float32)
                    + bl_ref[...][None, :])


_k4 = pl.pallas_call(
    _k4_body,
    out_shape=jax.ShapeDtypeStruct((GG, 1), jnp.float32),
)


def kernel(x, edge_index, batch, W1, b1, W2, b2, W3, b3, Wl, bl):
    src = edge_index[0].reshape(NT, NCH, CH)
    dst = edge_index[1].reshape(NT, NCH, CH)
    batch_ext = jnp.concatenate(
        [batch, jnp.full((NP - NN,), GG, dtype=batch.dtype)])
    deg16 = _deg_call(dst)
    g1, deg = _k1(deg16, x, W1)
    s1 = _agg_half(g1, src, dst)
    g2 = _k2(deg, s1, g1, b1, W2)
    s2 = _agg_half(g2, src, dst)
    g3 = _k3(deg, s2, g2, b2, W3)
    s3 = _agg_full(g3, src, dst)
    return _k4(deg, s3, g3, b3, batch_ext, Wl, bl)


# speedup vs baseline: 10.3516x; 1.2284x over previous
import functools

import jax
import jax.numpy as jnp
from jax import lax
from jax.experimental import pallas as pl
from jax.experimental.pallas import tpu as pltpu
from jax.experimental.pallas import tpu_sc as plsc

NN = 10000
NP = 10240
EE = 160000
GG = 64
INCH = 3000
HID = 256
OUT = 128
NT = 16
CH = 80
EPT = EE // NT
CAP = EPT + CH
NCMAX = CAP // CH
NG = EPT // 16
BN = 1024

_mesh = plsc.VectorSubcoreMesh(core_axis_name="c", subcore_axis_name="s")


def _dot3(a, b):
    return jnp.dot(a, b, preferred_element_type=jnp.float32)


def _prep_body(src_hbm, dst_hbm, deg_hbm, srcH_hbm, dstH_hbm,
               srcQ_hbm, dstQ_hbm, cnt_hbm,
               src_v, dst_v, cmpS, cmpD, cntb, degv):
    c = lax.axis_index("c")
    s = lax.axis_index("s")

    @pl.when(c == 0)
    def _():
        def zz(i, carry):
            degv[pl.ds(i * 16, 16)] = jnp.zeros((16,), jnp.float32)
            return carry

        lax.fori_loop(0, NP // 16, zz, 0)
        pltpu.sync_copy(dst_hbm.at[s], dst_v)
        ones = jnp.ones((16,), jnp.float32)

        def row(g, carry):
            idx = dst_v[pl.ds(16 * g, 16)]
            plsc.addupdate_scatter(degv, [idx], ones)
            return carry

        lax.fori_loop(0, NG, row, 0)
        pltpu.sync_copy(degv, deg_hbm.at[s])

    @pl.when(c == 1)
    def _():
        pltpu.sync_copy(src_hbm.at[s], src_v)
        pltpu.sync_copy(dst_hbm.at[s], dst_v)

        def do_list(l, lo, hr, oS, oD):
            def grp(g, cnt):
                d = dst_v[pl.ds(16 * g, 16)]
                sv = src_v[pl.ds(16 * g, 16)]
                m = (d >= lo) & (d < lo + hr)
                plsc.store_compressed(cmpD.at[pl.ds(cnt, 16)], d - lo, mask=m)
                plsc.store_compressed(cmpS.at[pl.ds(cnt, 16)], sv, mask=m)
                return cnt + jnp.max(plsc.all_reduce_population_count(m))

            cnt = lax.fori_loop(0, NG, grp, jnp.int32(0))
            dummyv = jnp.zeros((16,), jnp.int32) + (hr + (s & 7))
            for k in range(CH // 16):
                cmpS[pl.ds(cnt + 16 * k, 16)] = jnp.zeros((16,), jnp.int32)
                cmpD[pl.ds(cnt + 16 * k, 16)] = dummyv
            crow = cntb.at[l]
            crow[...] = jnp.zeros((16,), jnp.int32) + (cnt + CH - 1) // CH
            pltpu.sync_copy(cmpS, oS)
            pltpu.sync_copy(cmpD, oD)

        for p in range(2):
            do_list(p, 5120 * p, 5120, srcH_hbm.at[s, p], dstH_hbm.at[s, p])
        for q in range(4):
            do_list(2 + q, 2560 * q, 2560, srcQ_hbm.at[s, q], dstQ_hbm.at[s, q])
        pltpu.sync_copy(cntb, cnt_hbm.at[s])


_prep_call = pl.kernel(
    _prep_body,
    out_type=[
        jax.ShapeDtypeStruct((NT, NP), jnp.float32),
        jax.ShapeDtypeStruct((NT, 2, CAP), jnp.int32),
        jax.ShapeDtypeStruct((NT, 2, CAP), jnp.int32),
        jax.ShapeDtypeStruct((NT, 4, CAP), jnp.int32),
        jax.ShapeDtypeStruct((NT, 4, CAP), jnp.int32),
        jax.ShapeDtypeStruct((NT, 8, 16), jnp.int32),
    ],
    mesh=_mesh,
    scratch_types=[
        pltpu.VMEM((CAP,), jnp.int32),
        pltpu.VMEM((CAP,), jnp.int32),
        pltpu.VMEM((CAP,), jnp.int32),
        pltpu.VMEM((CAP,), jnp.int32),
        pltpu.VMEM((8, 16), jnp.int32),
        pltpu.VMEM((NP,), jnp.float32),
    ],
    compiler_params=pltpu.CompilerParams(needs_layout_passes=False),
)


def _agg_body(split, g_hbm, srcL_hbm, dstL_hbm, cnt_hbm, out_hbm,
              srcL, dstL, cntv, buf0, buf1, acc, sem0, sem1):
    c = lax.axis_index("c")
    s = lax.axis_index("s")
    hr = 5120 if split else 2560
    zr = hr // NT
    ghalf = g_hbm.at[c] if split else g_hbm

    pltpu.sync_copy(cnt_hbm.at[s], cntv)

    for p in (0, 1):
        if split:
            lo = hr * p
            srcsel = srcL_hbm.at[s, p]
            dstsel = dstL_hbm.at[s, p]
            n = cntv.at[p][pl.ds(0, 16)][0]
        else:
            q = 2 * c + p
            lo = hr * q
            srcsel = srcL_hbm.at[s, q]
            dstsel = dstL_hbm.at[s, q]
            n = cntv.at[2 + q][pl.ds(0, 16)][0]

        pltpu.sync_copy(srcsel, srcL)
        pltpu.sync_copy(dstsel, dstL)

        def zrow(i, carry):
            brow = buf0.at[i]
            for j in range(128 // 16):
                brow[pl.ds(j * 16, 16)] = jnp.zeros((16,), jnp.float32)
            return carry

        lax.fori_loop(0, CH, zrow, 0)
        for z in range(zr // CH):
            pltpu.sync_copy(buf0, acc.at[pl.ds(s * zr + z * CH, CH)])

        @pl.when(s < 8)
        def _():
            pltpu.sync_copy(buf0.at[pl.ds(0, 1)], acc.at[pl.ds(hr + s, 1)])

        plsc.subcore_barrier()

        def fire(j, buf, sem):
            pltpu.async_copy(ghalf.at[srcL.at[j]], buf, sem)

        def wait(buf, sem):
            pltpu.make_async_copy(ghalf.at[srcL.at[0]], buf, sem).wait()

        def scat(j, buf):
            pltpu.sync_copy(buf, acc.at[dstL.at[j]], add=True)

        @pl.when(n > 0)
        def _():
            fire(0, buf0, sem0)

        @pl.when(n > 1)
        def _():
            fire(1, buf1, sem1)

        def pair(i, carry):
            wait(buf0, sem0)
            scat(2 * i, buf0)

            @pl.when(2 * i + 2 < n)
            def _():
                fire(2 * i + 2, buf0, sem0)

            @pl.when(2 * i + 1 < n)
            def _():
                wait(buf1, sem1)
                scat(2 * i + 1, buf1)

                @pl.when(2 * i + 3 < n)
                def _():
                    fire(2 * i + 3, buf1, sem1)

            return carry

        lax.fori_loop(0, (n + 1) // 2, pair, 0)

        plsc.subcore_barrier()
        if split:
            odst = out_hbm.at[c, pl.ds(lo + s * zr, zr)]
        else:
            odst = out_hbm.at[pl.ds(lo + s * zr, zr)]
        pltpu.sync_copy(acc.at[pl.ds(s * zr, zr)], odst)
        plsc.subcore_barrier()


def _mk_agg(split):
    hr = 5120 if split else 2560
    oshape = (2, NP, 128) if split else (NP, 128)
    return pl.kernel(
        functools.partial(_agg_body, split),
        out_type=jax.ShapeDtypeStruct(oshape, jnp.float32),
        mesh=_mesh,
        scratch_types=[
            pltpu.VMEM((NCMAX, CH), jnp.int32),
            pltpu.VMEM((NCMAX, CH), jnp.int32),
            pltpu.VMEM((8, 16), jnp.int32),
            pltpu.VMEM((CH, 128), jnp.float32),
            pltpu.VMEM((CH, 128), jnp.float32),
            pltpu.VMEM_SHARED((hr + 8, 128), jnp.float32),
            pltpu.SemaphoreType.DMA,
            pltpu.SemaphoreType.DMA,
        ],
    )


_agg_half = _mk_agg(True)
_agg_full = _mk_agg(False)


def _k1_body(deg16_ref, x_ref, w_ref, g_ref, deg_ref):
    i = pl.program_id(0)
    deg = jnp.sum(deg16_ref[...], axis=0)
    deg_ref[...] = deg
    dinv = lax.rsqrt(deg + 1.0)[:, None]
    t = _dot3(x_ref[...], w_ref[...])
    row = i * BN + lax.broadcasted_iota(jnp.int32, (BN, 1), 0)
    g = jnp.where(row < NN, t * dinv, 0.0)
    g_ref[0] = g[:, :128]
    g_ref[1] = g[:, 128:]


_k1 = pl.pallas_call(
    _k1_body,
    grid=(NP // BN,),
    in_specs=[
        pl.BlockSpec((NT, BN), lambda i: (0, i)),
        pl.BlockSpec((BN, INCH), lambda i: (i, 0)),
        pl.BlockSpec((INCH, HID), lambda i: (0, 0)),
    ],
    out_specs=[
        pl.BlockSpec((2, BN, 128), lambda i: (0, i, 0)),
        pl.BlockSpec((BN,), lambda i: (i,)),
    ],
    out_shape=[
        jax.ShapeDtypeStruct((2, NP, 128), jnp.float32),
        jax.ShapeDtypeStruct((NP,), jnp.float32),
    ],
)


def _k2_body(deg_ref, s_ref, g_ref, b_ref, w_ref, out_ref):
    dinv = lax.rsqrt(deg_ref[...] + 1.0)[:, None]
    h = jnp.concatenate([s_ref[0] + g_ref[0], s_ref[1] + g_ref[1]], axis=1)
    h = jnp.maximum(h * dinv + b_ref[...][None, :], 0.0)
    t = _dot3(h, w_ref[...])
    g = t * dinv
    out_ref[0] = g[:, :128]
    out_ref[1] = g[:, 128:]


_k2 = pl.pallas_call(
    _k2_body,
    grid=(NP // BN,),
    in_specs=[
        pl.BlockSpec((BN,), lambda i: (i,)),
        pl.BlockSpec((2, BN, 128), lambda i: (0, i, 0)),
        pl.BlockSpec((2, BN, 128), lambda i: (0, i, 0)),
        pl.BlockSpec((HID,), lambda i: (0,)),
        pl.BlockSpec((HID, HID), lambda i: (0, 0)),
    ],
    out_specs=pl.BlockSpec((2, BN, 128), lambda i: (0, i, 0)),
    out_shape=jax.ShapeDtypeStruct((2, NP, 128), jnp.float32),
)


def _k3_body(deg_ref, s_ref, g_ref, b_ref, w_ref, out_ref):
    dinv = lax.rsqrt(deg_ref[...] + 1.0)[:, None]
    h = jnp.concatenate([s_ref[0] + g_ref[0], s_ref[1] + g_ref[1]], axis=1)
    h = jnp.maximum(h * dinv + b_ref[...][None, :], 0.0)
    t = _dot3(h, w_ref[...])
    out_ref[...] = t * dinv


_k3 = pl.pallas_call(
    _k3_body,
    grid=(NP // BN,),
    in_specs=[
        pl.BlockSpec((BN,), lambda i: (i,)),
        pl.BlockSpec((2, BN, 128), lambda i: (0, i, 0)),
        pl.BlockSpec((2, BN, 128), lambda i: (0, i, 0)),
        pl.BlockSpec((HID,), lambda i: (0,)),
        pl.BlockSpec((HID, OUT), lambda i: (0, 0)),
    ],
    out_specs=pl.BlockSpec((BN, OUT), lambda i: (i, 0)),
    out_shape=jax.ShapeDtypeStruct((NP, OUT), jnp.float32),
)


def _k4_body(deg_ref, s_ref, g_ref, b_ref, batch_ref, wl_ref, bl_ref, out_ref):
    dinv = lax.rsqrt(deg_ref[...] + 1.0)[:, None]
    h = jnp.maximum((s_ref[...] + g_ref[...]) * dinv + b_ref[...][None, :],
                    0.0)
    seg = lax.broadcasted_iota(jnp.int32, (GG, NP), 0)
    P = (batch_ref[...][None, :] == seg).astype(jnp.float32)
    sums = jnp.dot(P, h, preferred_element_type=jnp.float32)
    counts = jnp.sum(P, axis=1, keepdims=True)
    pooled = sums / jnp.maximum(counts, 1.0)
    out_ref[...] = (jnp.dot(pooled, wl_ref[...],
                        preferred_element_type=jnp.float32)
                    + bl_ref[...][None, :])


_k4 = pl.pallas_call(
    _k4_body,
    out_shape=jax.ShapeDtypeStruct((GG, 1), jnp.float32),
)


def kernel(x, edge_index, batch, W1, b1, W2, b2, W3, b3, Wl, bl):
    src = jnp.pad(edge_index[0].reshape(NT, EPT), ((0, 0), (0, CH)))
    dst = jnp.pad(edge_index[1].reshape(NT, EPT), ((0, 0), (0, CH)))
    batch_ext = jnp.concatenate(
        [batch, jnp.full((NP - NN,), GG, dtype=batch.dtype)])
    deg16, srcH, dstH, srcQ, dstQ, cnts = _prep_call(src, dst)
    srcH = srcH.reshape(NT, 2, NCMAX, CH)
    dstH = dstH.reshape(NT, 2, NCMAX, CH)
    srcQ = srcQ.reshape(NT, 4, NCMAX, CH)
    dstQ = dstQ.reshape(NT, 4, NCMAX, CH)
    g1, deg = _k1(deg16, x, W1)
    s1 = _agg_half(g1, srcH, dstH, cnts)
    g2 = _k2(deg, s1, g1, b1, W2)
    s2 = _agg_half(g2, srcH, dstH, cnts)
    g3 = _k3(deg, s2, g2, b2, W3)
    s3 = _agg_full(g3, srcQ, dstQ, cnts)
    return _k4(deg, s3, g3, b3, batch_ext, Wl, bl)
